# Initial kernel scaffold; baseline (speedup 1.0000x reference)
import functools

import jax
import jax.numpy as jnp
from jax import lax
from jax.experimental import pallas as pl
from jax.experimental.pallas import tpu as pltpu
from jax.experimental.pallas import tpu_sc as plsc

N = 10000
E = 320000
D = 128

NC = 2
NS = 16
NW = NC * NS
L = 16

CHUNK = 128
NCHUNK = E // CHUNK
BASE_K = NCHUNK // NW
EXTRA = NCHUNK - BASE_K * NW

ROWS_PER_TILE = N // NS
ZB = 125


def _pre_body(x_ref, w1_ref, b1_ref, a_ref, b_ref):
    x = x_ref[...]
    w1 = w1_ref[...]
    a_ref[...] = (
        jnp.dot(x, w1[:D, :], preferred_element_type=jnp.float32) + b1_ref[...]
    )
    b_ref[...] = jnp.dot(x, w1[D:, :], preferred_element_type=jnp.float32)


def _pre(x, w1, b1_row):
    blk = 1000
    grid = N // blk
    return pl.pallas_call(
        _pre_body,
        grid=(grid,),
        in_specs=[
            pl.BlockSpec((blk, D), lambda i: (i, 0)),
            pl.BlockSpec((2 * D, D), lambda i: (0, 0)),
            pl.BlockSpec((1, D), lambda i: (0, 0)),
        ],
        out_specs=[
            pl.BlockSpec((blk, D), lambda i: (i, 0)),
            pl.BlockSpec((blk, D), lambda i: (i, 0)),
        ],
        out_shape=[
            jax.ShapeDtypeStruct((N, D), jnp.float32),
            jax.ShapeDtypeStruct((N, D), jnp.float32),
        ],
    )(x, w1, b1_row)


_MESH = plsc.VectorSubcoreMesh(core_axis_name="c", subcore_axis_name="s")


@functools.partial(
    pl.kernel,
    out_type=(
        jax.ShapeDtypeStruct((NC, N, D), jnp.float32),
        jax.ShapeDtypeStruct((NC, N, L), jnp.float32),
    ),
    mesh=_MESH,
    scratch_types=[
        pltpu.VMEM((CHUNK,), jnp.int32),
        pltpu.VMEM((CHUNK,), jnp.int32),
        pltpu.VMEM((CHUNK, D), jnp.float32),
        pltpu.VMEM((CHUNK, D), jnp.float32),
        pltpu.VMEM((CHUNK, L), jnp.float32),
        pltpu.VMEM((ZB, D), jnp.float32),
        pltpu.VMEM((ZB, L), jnp.float32),
        pltpu.VMEM_SHARED((N, D), jnp.float32),
        pltpu.VMEM_SHARED((N, L), jnp.float32),
        pltpu.SemaphoreType.DMA,
        pltpu.SemaphoreType.DMA,
    ],
)
def _edge_kernel(
    src_hbm, dst_hbm, a_hbm, b_hbm,
    h_out, cnt_out,
    src_v, dst_v, a_v, b_v, ones_v, z_v, z16_v,
    h_sh, cnt_sh, sem_a, sem_b,
):
    cid = lax.axis_index("c")
    sid = lax.axis_index("s")
    w = sid * NC + cid

    zero16 = jnp.zeros((L,), jnp.float32)
    one16 = jnp.ones((L,), jnp.float32)

    def fill_body(i, _):
        for l in range(D // L):
            z_v[i, pl.ds(l * L, L)] = zero16
        z16_v[i, pl.ds(0, L)] = zero16
        return 0

    lax.fori_loop(0, ZB, fill_body, 0)

    def ones_body(i, _):
        ones_v[i, pl.ds(0, L)] = one16
        return 0

    lax.fori_loop(0, CHUNK, ones_body, 0)

    row0 = sid * ROWS_PER_TILE
    for q in range(ROWS_PER_TILE // ZB):
        pltpu.sync_copy(z_v, h_sh.at[pl.ds(row0 + q * ZB, ZB)])
        pltpu.sync_copy(z16_v, cnt_sh.at[pl.ds(row0 + q * ZB, ZB)])
    plsc.subcore_barrier()

    nk = jnp.where(w < EXTRA, BASE_K + 1, BASE_K)

    def chunk_body(k, _):
        base = (w + k * NW) * CHUNK
        pltpu.sync_copy(src_hbm.at[pl.ds(base, CHUNK)], src_v)
        pltpu.sync_copy(dst_hbm.at[pl.ds(base, CHUNK)], dst_v)
        cp_a = pltpu.async_copy(a_hbm.at[dst_v], a_v, sem_a)
        cp_b = pltpu.async_copy(b_hbm.at[src_v], b_v, sem_b)
        cp_a.wait()
        cp_b.wait()

        def row_body(i, _):
            for l in range(D // L):
                av = a_v[i, pl.ds(l * L, L)]
                bv = b_v[i, pl.ds(l * L, L)]
                a_v[i, pl.ds(l * L, L)] = jnp.maximum(av + bv, 0.0)
            return 0

        lax.fori_loop(0, CHUNK, row_body, 0)

        pltpu.sync_copy(a_v, h_sh.at[dst_v], add=True)
        pltpu.sync_copy(ones_v, cnt_sh.at[dst_v], add=True)
        return 0

    lax.fori_loop(0, nk, chunk_body, 0)

    plsc.subcore_barrier()
    pltpu.sync_copy(
        h_sh.at[pl.ds(row0, ROWS_PER_TILE)],
        h_out.at[cid, pl.ds(row0, ROWS_PER_TILE)],
    )
    pltpu.sync_copy(
        cnt_sh.at[pl.ds(row0, ROWS_PER_TILE)],
        cnt_out.at[cid, pl.ds(row0, ROWS_PER_TILE)],
    )


def _post_body(x_ref, hp_ref, cnt_ref, w2_ref, b2_ref, w3_ref, b3_ref,
               w4_ref, b4_ref, out_ref):
    h = hp_ref[0] + hp_ref[1]
    cnt = cnt_ref[0, :, 0:1] + cnt_ref[1, :, 0:1]
    aggr = (
        jnp.dot(h, w2_ref[...], preferred_element_type=jnp.float32)
        + cnt * b2_ref[...]
    )
    w3 = w3_ref[...]
    u = jnp.maximum(
        jnp.dot(x_ref[...], w3[:D, :], preferred_element_type=jnp.float32)
        + jnp.dot(aggr, w3[D:, :], preferred_element_type=jnp.float32)
        + b3_ref[...],
        0.0,
    )
    out_ref[...] = (
        jnp.dot(u, w4_ref[...], preferred_element_type=jnp.float32)
        + b4_ref[...]
    )


def _post(x, hp, cnt, w2, b2_row, w3, b3_row, w4, b4_row):
    blk = 1000
    grid = N // blk
    return pl.pallas_call(
        _post_body,
        grid=(grid,),
        in_specs=[
            pl.BlockSpec((blk, D), lambda i: (i, 0)),
            pl.BlockSpec((NC, blk, D), lambda i: (0, i, 0)),
            pl.BlockSpec((NC, blk, L), lambda i: (0, i, 0)),
            pl.BlockSpec((D, D), lambda i: (0, 0)),
            pl.BlockSpec((1, D), lambda i: (0, 0)),
            pl.BlockSpec((2 * D, D), lambda i: (0, 0)),
            pl.BlockSpec((1, D), lambda i: (0, 0)),
            pl.BlockSpec((D, D), lambda i: (0, 0)),
            pl.BlockSpec((1, D), lambda i: (0, 0)),
        ],
        out_specs=pl.BlockSpec((blk, D), lambda i: (i, 0)),
        out_shape=jax.ShapeDtypeStruct((N, D), jnp.float32),
    )(x, hp, cnt, w2, b2_row, w3, b3_row, w4, b4_row)


def kernel(x, edge_index, W1, b1, W2, b2, W3, b3, W4, b4):
    src = edge_index[0]
    dst = edge_index[1]
    a, b = _pre(x, W1, b1.reshape(1, D))
    hp, cnt = _edge_kernel(src, dst, a, b)
    return _post(
        x, hp, cnt, W2, b2.reshape(1, D), W3, b3.reshape(1, D),
        W4, b4.reshape(1, D),
    )


# speedup vs baseline: 4.5577x; 4.5577x over previous
import functools

import jax
import jax.numpy as jnp
from jax import lax
from jax.experimental import pallas as pl
from jax.experimental.pallas import tpu as pltpu
from jax.experimental.pallas import tpu_sc as plsc

N = 10000
E = 320000
D = 128
DH = D // 2

NC = 2
NS = 16
L = 16

CHUNK = 128
NCHUNK = E // CHUNK
BASE_K = NCHUNK // NS
EXTRA = NCHUNK - BASE_K * NS

DRAIN_TILES = 10
DRAIN_ROWS = N // DRAIN_TILES
ZB = 125


def _pre_body(x_ref, w1_ref, b1_ref, a0_ref, a1_ref, b0_ref, b1h_ref):
    x = x_ref[...]
    w1 = w1_ref[...]
    a = jnp.dot(x, w1[:D, :], preferred_element_type=jnp.float32) + b1_ref[...]
    b = jnp.dot(x, w1[D:, :], preferred_element_type=jnp.float32)
    a0_ref[...] = a[:, :DH]
    a1_ref[...] = a[:, DH:]
    b0_ref[...] = b[:, :DH]
    b1h_ref[...] = b[:, DH:]


def _pre(x, w1, b1_row):
    blk = 1000
    grid = N // blk
    half = jax.ShapeDtypeStruct((N, DH), jnp.float32)
    return pl.pallas_call(
        _pre_body,
        grid=(grid,),
        in_specs=[
            pl.BlockSpec((blk, D), lambda i: (i, 0)),
            pl.BlockSpec((2 * D, D), lambda i: (0, 0)),
            pl.BlockSpec((1, D), lambda i: (0, 0)),
        ],
        out_specs=[pl.BlockSpec((blk, DH), lambda i: (i, 0))] * 4,
        out_shape=[half, half, half, half],
    )(x, w1, b1_row)


_MESH = plsc.VectorSubcoreMesh(core_axis_name="c", subcore_axis_name="s")


@functools.partial(
    pl.kernel,
    out_type=(
        jax.ShapeDtypeStruct((NC, N, DH), jnp.float32),
        jax.ShapeDtypeStruct((N, L), jnp.float32),
    ),
    mesh=_MESH,
    compiler_params=pltpu.CompilerParams(use_tc_tiling_on_sc=False),
    scratch_types=[
        pltpu.VMEM((CHUNK,), jnp.int32),
        pltpu.VMEM((CHUNK,), jnp.int32),
        pltpu.VMEM((CHUNK, DH), jnp.float32),
        pltpu.VMEM((CHUNK, DH), jnp.float32),
        pltpu.VMEM((CHUNK, L), jnp.float32),
        pltpu.VMEM((ZB, DH), jnp.float32),
        pltpu.VMEM((ZB, L), jnp.float32),
        pltpu.VMEM_SHARED((N, DH), jnp.float32),
        pltpu.VMEM_SHARED((N, L), jnp.float32),
        pltpu.SemaphoreType.DMA,
        pltpu.SemaphoreType.DMA,
    ],
)
def _edge_kernel(
    src_hbm, dst_hbm, a_hbm, b_hbm,
    h_out, cnt_out,
    src_v, dst_v, a_v, b_v, ones_v, z_v, z16_v,
    h_sh, cnt_sh, sem_a, sem_b,
):
    cid = lax.axis_index("c")
    sid = lax.axis_index("s")

    zero16 = jnp.zeros((L,), jnp.float32)
    one16 = jnp.ones((L,), jnp.float32)

    def fill_body(i, _):
        for l in range(DH // L):
            z_v[i, pl.ds(l * L, L)] = zero16
        z16_v[i, pl.ds(0, L)] = zero16
        return 0

    lax.fori_loop(0, ZB, fill_body, 0)

    def ones_body(i, _):
        ones_v[i, pl.ds(0, L)] = one16
        return 0

    lax.fori_loop(0, CHUNK, ones_body, 0)

    row0 = sid * DRAIN_ROWS

    @pl.when(sid < DRAIN_TILES)
    def _zero():
        for q in range(DRAIN_ROWS // ZB):
            pltpu.sync_copy(z_v, h_sh.at[pl.ds(row0 + q * ZB, ZB)])
            pltpu.sync_copy(z16_v, cnt_sh.at[pl.ds(row0 + q * ZB, ZB)])

    plsc.subcore_barrier()

    nk = jnp.where(sid < EXTRA, BASE_K + 1, BASE_K)
    is_core0 = cid == 0

    def chunk_body(k, _):
        base = (sid + k * NS) * CHUNK
        pltpu.sync_copy(src_hbm.at[pl.ds(base, CHUNK)], src_v)
        pltpu.sync_copy(dst_hbm.at[pl.ds(base, CHUNK)], dst_v)
        cp_a = pltpu.async_copy(a_hbm.at[cid].at[dst_v], a_v, sem_a)
        cp_b = pltpu.async_copy(b_hbm.at[cid].at[src_v], b_v, sem_b)
        cp_a.wait()
        cp_b.wait()

        def row_body(i, _):
            for l in range(DH // L):
                av = a_v[i, pl.ds(l * L, L)]
                bv = b_v[i, pl.ds(l * L, L)]
                a_v[i, pl.ds(l * L, L)] = jnp.maximum(av + bv, 0.0)
            return 0

        lax.fori_loop(0, CHUNK, row_body, 0)

        pltpu.sync_copy(a_v, h_sh.at[dst_v], add=True)

        @pl.when(is_core0)
        def _count():
            pltpu.sync_copy(ones_v, cnt_sh.at[dst_v], add=True)

        return 0

    lax.fori_loop(0, nk, chunk_body, 0)

    plsc.subcore_barrier()

    @pl.when(sid < DRAIN_TILES)
    def _drain():
        pltpu.sync_copy(
            h_sh.at[pl.ds(row0, DRAIN_ROWS)],
            h_out.at[cid, pl.ds(row0, DRAIN_ROWS)],
        )

        @pl.when(is_core0)
        def _drain_cnt():
            pltpu.sync_copy(
                cnt_sh.at[pl.ds(row0, DRAIN_ROWS)],
                cnt_out.at[pl.ds(row0, DRAIN_ROWS)],
            )


def _post_body(x_ref, hp_ref, cnt_ref, w2_ref, b2_ref, w3_ref, b3_ref,
               w4_ref, b4_ref, out_ref):
    w2 = w2_ref[...]
    cnt = cnt_ref[:, 0:1]
    aggr = (
        jnp.dot(hp_ref[0], w2[:DH, :], preferred_element_type=jnp.float32)
        + jnp.dot(hp_ref[1], w2[DH:, :], preferred_element_type=jnp.float32)
        + cnt * b2_ref[...]
    )
    w3 = w3_ref[...]
    u = jnp.maximum(
        jnp.dot(x_ref[...], w3[:D, :], preferred_element_type=jnp.float32)
        + jnp.dot(aggr, w3[D:, :], preferred_element_type=jnp.float32)
        + b3_ref[...],
        0.0,
    )
    out_ref[...] = (
        jnp.dot(u, w4_ref[...], preferred_element_type=jnp.float32)
        + b4_ref[...]
    )


def _post(x, hp, cnt, w2, b2_row, w3, b3_row, w4, b4_row):
    blk = 1000
    grid = N // blk
    return pl.pallas_call(
        _post_body,
        grid=(grid,),
        in_specs=[
            pl.BlockSpec((blk, D), lambda i: (i, 0)),
            pl.BlockSpec((NC, blk, DH), lambda i: (0, i, 0)),
            pl.BlockSpec((blk, L), lambda i: (i, 0)),
            pl.BlockSpec((D, D), lambda i: (0, 0)),
            pl.BlockSpec((1, D), lambda i: (0, 0)),
            pl.BlockSpec((2 * D, D), lambda i: (0, 0)),
            pl.BlockSpec((1, D), lambda i: (0, 0)),
            pl.BlockSpec((D, D), lambda i: (0, 0)),
            pl.BlockSpec((1, D), lambda i: (0, 0)),
        ],
        out_specs=pl.BlockSpec((blk, D), lambda i: (i, 0)),
        out_shape=jax.ShapeDtypeStruct((N, D), jnp.float32),
    )(x, hp, cnt, w2, b2_row, w3, b3_row, w4, b4_row)


def kernel(x, edge_index, W1, b1, W2, b2, W3, b3, W4, b4):
    src = edge_index[0]
    dst = edge_index[1]
    a0, a1, b0, b1h = _pre(x, W1, b1.reshape(1, D))
    a_halves = jnp.stack([a0, a1])
    b_halves = jnp.stack([b0, b1h])
    hp, cnt = _edge_kernel(src, dst, a_halves, b_halves)
    return _post(
        x, hp, cnt, W2, b2.reshape(1, D), W3, b3.reshape(1, D),
        W4, b4.reshape(1, D),
    )
